# baseline (speedup 1.0000x reference)
import functools

import jax
import jax.numpy as jnp
from jax import lax
from jax.experimental import pallas as pl
from jax.experimental.pallas import tpu as pltpu
from jax.experimental.pallas import tpu_sc as plsc

NC, NS = 2, 16
NW = NC * NS
BB = 16384
DD = 64
BPW = BB // NW
CH = 128
NCH = BPW // CH

WS = 0.7
WR = 0.3
EPS = 0.5
REG = 0.001

def _sc_gather_body(rows_hbm, cols_hbm, p_hbm, q_hbm, z_hbm,
                    ues_out, urat_out, urel_out,
                    ridx, cidx, p_v, q_v, z_v, sem):
    wid = lax.axis_index("s") * NC + lax.axis_index("c")
    base = wid * BPW
    for k in range(NCH):
        pltpu.sync_copy(rows_hbm.at[pl.ds(base + k * CH, CH)], ridx.at[k])
        pltpu.sync_copy(cols_hbm.at[pl.ds(base + k * CH, CH)], cidx.at[k])
    copies = []
    for k in range(NCH):
        sl = pl.ds(k * CH, CH)
        copies.append(pltpu.async_copy(p_hbm.at[ridx.at[k]], p_v.at[sl], sem))
        copies.append(pltpu.async_copy(q_hbm.at[cidx.at[k]], q_v.at[sl], sem))
        copies.append(pltpu.async_copy(z_hbm.at[cidx.at[k]], z_v.at[sl], sem))
    for c in copies:
        c.wait()
    pltpu.sync_copy(p_v, ues_out.at[pl.ds(base, BPW)])
    pltpu.sync_copy(q_v, urat_out.at[pl.ds(base, BPW)])
    pltpu.sync_copy(z_v, urel_out.at[pl.ds(base, BPW)])


@functools.cache
def _sc_gather_kernel():
    mesh = plsc.VectorSubcoreMesh(
        core_axis_name="c", subcore_axis_name="s", num_cores=NC, num_subcores=NS
    )
    return pl.kernel(
        _sc_gather_body,
        out_type=[jax.ShapeDtypeStruct((BB, DD), jnp.float32)] * 3,
        mesh=mesh,
        scratch_types=[
            pltpu.VMEM((NCH, CH), jnp.int32),
            pltpu.VMEM((NCH, CH), jnp.int32),
            pltpu.VMEM((BPW, DD), jnp.float32),
            pltpu.VMEM((BPW, DD), jnp.float32),
            pltpu.VMEM((BPW, DD), jnp.float32),
            pltpu.SemaphoreType.DMA,
        ],
        compiler_params=pltpu.CompilerParams(use_tc_tiling_on_sc=False),
    )


_RB = 2048


def _loss_body(ues_ref, urat_ref, urel_ref, rat_ref, rel_ref, sen_ref,
               senh_ref, out_ref):
    i = pl.program_id(0)
    ues = ues_ref[...]
    urat = urat_ref[...]
    urel = urel_ref[...]
    ratval = rat_ref[...]
    relval = rel_ref[...]
    senval = sen_ref[...]
    senh = senh_ref[...] - 0.0001
    relv = relval - 0.0001
    pos = relv != 0.0
    neg = (relv - 1.0) != 0.0

    pr_rat = jnp.sum(ues * urat, axis=1, keepdims=True)
    pr_rel = jnp.sum(ues * urel, axis=1, keepdims=True)
    s_rat = 1.0 / (1.0 + jnp.exp(-pr_rat))
    s_rel = 1.0 / (1.0 + jnp.exp(-pr_rel))

    loss_rel = (jnp.sum(jnp.where(pos, jnp.log(s_rel), 0.0))
                + jnp.sum(jnp.where(neg, jnp.log(1.0 - s_rel), 0.0)))

    sen_rel_val = jnp.where(senh >= EPS, 1.0, 0.0)
    flag = sen_rel_val + relv
    rel_rel = jnp.where(flag == 2.0, 1.0, 0.0)
    sen_w = sen_rel_val + (WS - 1.0) * rel_rel
    rat_w = relv + (-WS) * rel_rel

    loss_rat = jnp.sum((s_rat - ratval) ** 2 * rat_w)
    loss_sen = jnp.sum((s_rat - senval) ** 2 * sen_w)
    ssq = jnp.sum(ues * ues) + jnp.sum(urat * urat) + jnp.sum(urel * urel)

    part = loss_sen + loss_rat - WR * loss_rel + REG * ssq

    @pl.when(i == 0)
    def _():
        out_ref[...] = jnp.zeros_like(out_ref)

    out_ref[...] += jnp.full((1, 1), part, dtype=jnp.float32)


def _tc_loss(ues, urat, urel, ratval, relval, senval, senh):
    mat_spec = pl.BlockSpec((_RB, DD), lambda i: (i, 0))
    vec_spec = pl.BlockSpec((_RB, 1), lambda i: (i, 0))
    return pl.pallas_call(
        _loss_body,
        grid=(BB // _RB,),
        in_specs=[mat_spec] * 3 + [vec_spec] * 4,
        out_specs=pl.BlockSpec((1, 1), lambda i: (0, 0)),
        out_shape=jax.ShapeDtypeStruct((1, 1), jnp.float32),
    )(ues, urat, urel, ratval, relval, senval, senh)


def kernel(rows, cols, ratval, relval, senval, senhelval, P, Q, Z):
    ues, urat, urel = _sc_gather_kernel()(rows, cols, P, Q, Z)
    loss = _tc_loss(
        ues, urat, urel,
        ratval.reshape(BB, 1), relval.reshape(BB, 1),
        senval.reshape(BB, 1), senhelval.reshape(BB, 1),
    )
    return loss[0, 0]


# speedup vs baseline: 1.0062x; 1.0062x over previous
import functools

import jax
import jax.numpy as jnp
from jax import lax
from jax.experimental import pallas as pl
from jax.experimental.pallas import tpu as pltpu
from jax.experimental.pallas import tpu_sc as plsc

NC, NS = 2, 16
NW = NC * NS
BB = 16384
DD = 64
BPW = BB // NW
CH = 128
NCH = BPW // CH

WS = 0.7
WR = 0.3
EPS = 0.5
REG = 0.001


def _sc_gather_body(rows_hbm, cols_hbm, p_hbm, q_hbm, z_hbm,
                    ues_out, urat_out, urel_out,
                    ridx, cidx, p_v, q_v, z_v, sem):
    wid = lax.axis_index("s") * NC + lax.axis_index("c")
    base = wid * BPW
    for k in range(NCH):
        pltpu.sync_copy(rows_hbm.at[pl.ds(base + k * CH, CH)], ridx.at[k])
        pltpu.sync_copy(cols_hbm.at[pl.ds(base + k * CH, CH)], cidx.at[k])
    copies = []
    for k in range(NCH):
        sl = pl.ds(k * CH, CH)
        copies.append(pltpu.async_copy(p_hbm.at[ridx.at[k]], p_v.at[sl], sem))
        copies.append(pltpu.async_copy(q_hbm.at[cidx.at[k]], q_v.at[sl], sem))
        copies.append(pltpu.async_copy(z_hbm.at[cidx.at[k]], z_v.at[sl], sem))
    for c in copies:
        c.wait()
    pltpu.sync_copy(p_v, ues_out.at[pl.ds(base, BPW)])
    pltpu.sync_copy(q_v, urat_out.at[pl.ds(base, BPW)])
    pltpu.sync_copy(z_v, urel_out.at[pl.ds(base, BPW)])


@functools.cache
def _sc_gather_kernel():
    mesh = plsc.VectorSubcoreMesh(
        core_axis_name="c", subcore_axis_name="s", num_cores=NC, num_subcores=NS
    )
    return pl.kernel(
        _sc_gather_body,
        out_type=[jax.ShapeDtypeStruct((BB, DD), jnp.float32)] * 3,
        mesh=mesh,
        scratch_types=[
            pltpu.VMEM((NCH, CH), jnp.int32),
            pltpu.VMEM((NCH, CH), jnp.int32),
            pltpu.VMEM((BPW, DD), jnp.float32),
            pltpu.VMEM((BPW, DD), jnp.float32),
            pltpu.VMEM((BPW, DD), jnp.float32),
            pltpu.SemaphoreType.DMA,
        ],
        compiler_params=pltpu.CompilerParams(use_tc_tiling_on_sc=False),
    )


_RB = 2048
_VR = _RB // 128


def _loss_body(ues_ref, urat_ref, urel_ref, rat_ref, rel_ref, sen_ref,
               senh_ref, out_ref):
    i = pl.program_id(0)
    ues = ues_ref[...]
    urat = urat_ref[...]
    urel = urel_ref[...]
    ratval = rat_ref[...]
    relval = rel_ref[...]
    senval = sen_ref[...]
    senh = senh_ref[...] - 0.0001
    relv = relval - 0.0001
    pos = relv != 0.0
    neg = (relv - 1.0) != 0.0

    pr_rat = jnp.sum(ues * urat, axis=2)
    pr_rel = jnp.sum(ues * urel, axis=2)
    s_rat = 1.0 / (1.0 + jnp.exp(-pr_rat))
    s_rel = 1.0 / (1.0 + jnp.exp(-pr_rel))

    loss_rel = (jnp.sum(jnp.where(pos, jnp.log(s_rel), 0.0))
                + jnp.sum(jnp.where(neg, jnp.log(1.0 - s_rel), 0.0)))

    sen_rel_val = jnp.where(senh >= EPS, 1.0, 0.0)
    flag = sen_rel_val + relv
    rel_rel = jnp.where(flag == 2.0, 1.0, 0.0)
    sen_w = sen_rel_val + (WS - 1.0) * rel_rel
    rat_w = relv + (-WS) * rel_rel

    loss_rat = jnp.sum((s_rat - ratval) ** 2 * rat_w)
    loss_sen = jnp.sum((s_rat - senval) ** 2 * sen_w)
    ssq = jnp.sum(ues * ues) + jnp.sum(urat * urat) + jnp.sum(urel * urel)

    part = loss_sen + loss_rat - WR * loss_rel + REG * ssq

    @pl.when(i == 0)
    def _():
        out_ref[...] = jnp.zeros_like(out_ref)

    out_ref[...] += jnp.full((1, 1), part, dtype=jnp.float32)


def _tc_loss(ues, urat, urel, ratval, relval, senval, senh):
    mat_spec = pl.BlockSpec((_VR, 128, DD), lambda i: (i, 0, 0))
    vec_spec = pl.BlockSpec((_VR, 128), lambda i: (i, 0))
    return pl.pallas_call(
        _loss_body,
        grid=(BB // _RB,),
        in_specs=[mat_spec] * 3 + [vec_spec] * 4,
        out_specs=pl.BlockSpec((1, 1), lambda i: (0, 0)),
        out_shape=jax.ShapeDtypeStruct((1, 1), jnp.float32),
    )(ues, urat, urel, ratval, relval, senval, senh)


def kernel(rows, cols, ratval, relval, senval, senhelval, P, Q, Z):
    ues, urat, urel = _sc_gather_kernel()(rows, cols, P, Q, Z)
    loss = _tc_loss(
        ues.reshape(128, 128, DD), urat.reshape(128, 128, DD),
        urel.reshape(128, 128, DD),
        ratval.reshape(128, 128), relval.reshape(128, 128),
        senval.reshape(128, 128), senhelval.reshape(128, 128),
    )
    return loss[0, 0]


# speedup vs baseline: 1.9018x; 1.8900x over previous
import functools

import jax
import jax.numpy as jnp
import numpy as np
from jax import lax
from jax.experimental import pallas as pl
from jax.experimental.pallas import tpu as pltpu
from jax.experimental.pallas import tpu_sc as plsc

NC, NS = 2, 16
NW = NC * NS
BB = 16384
DD = 64
BPW = BB // NW
CHC = 128
NCHK = BPW // CHC
NPB = 7813
PTAIL = (NPB - 1) * 128

WS = 0.7
WR = 0.3
EPS = 0.5
REG = 0.001


NBUF = 8


def _sc_body(rows_hbm, cols_hbm, pt_hbm, q2_hbm, z2_hbm, ptail_hbm,
             drat_out, drel_out, ssq_out,
             rows_v, cols_v, qidx, tailbuf, blk, qraw, zraw, p_v,
             drat_v, drel_v, ssq_v, sem, *bsems):
    wid = lax.axis_index("s") * NC + lax.axis_index("c")
    base = wid * BPW
    lane = lax.iota(jnp.int32, 16)
    pltpu.sync_copy(rows_hbm.at[pl.ds(base, BPW)], rows_v)
    pltpu.sync_copy(cols_hbm.at[pl.ds(base, BPW)], cols_v)
    pltpu.sync_copy(ptail_hbm, tailbuf)

    def qidx_body(v, carry):
        s = pl.multiple_of(v * 16, 16)
        cv = cols_v[pl.ds(s, 16)]
        k = v // (CHC // 16)
        m = v % (CHC // 16)
        qidx[k, pl.ds(m * 16, 16)] = lax.shift_right_logical(cv, 1)
        return carry

    lax.fori_loop(0, BPW // 16, qidx_body, 0)

    def issue_blk(rvec, h, slot):
        r = rvec[h]
        t = jnp.minimum(lax.shift_right_logical(r, 7), jnp.int32(NPB - 2))
        pltpu.async_copy(pt_hbm.at[:, pl.ds(t * 128, 128)], blk.at[slot],
                         bsems[slot])

    def extract_blk(rvec, h, slot, jcol):
        pltpu.make_async_copy(pt_hbm.at[:, pl.ds(0, 128)], blk.at[slot],
                              bsems[slot]).wait()
        r = rvec[h]
        is_tail = r >= PTAIL
        lm = jnp.where(is_tail, jnp.int32(0), jnp.bitwise_and(r, 127))
        lt = jnp.where(is_tail, r - PTAIL, jnp.int32(0))
        col = jnp.broadcast_to(jcol + h, (16,)).astype(jnp.int32)
        tmask = jnp.broadcast_to(is_tail, (16,))
        for gg in range(DD // 16):
            fidx = lane + (gg * 16)
            vals = plsc.load_gather(
                blk.at[slot], [fidx, jnp.broadcast_to(lm, (16,))])
            tvals = plsc.load_gather(
                tailbuf, [fidx, jnp.broadcast_to(lt, (16,))])
            plsc.store_scatter(p_v, [fidx, col],
                               jnp.where(tmask, tvals, vals))

    def chunk_body(k, ssq_in):
        qcp = pltpu.async_copy(q2_hbm.at[qidx.at[k]], qraw, sem)
        zcp = pltpu.async_copy(z2_hbm.at[qidx.at[k]], zraw, sem)

        def grp_body(gv, carry):
            s16 = pl.multiple_of(k * CHC + gv * 16, 16)
            rvec = rows_v[pl.ds(s16, 16)]
            jcol = gv * 16
            for h in range(NBUF):
                issue_blk(rvec, h, h)
            for h in range(NBUF):
                extract_blk(rvec, h, h, jcol)
                issue_blk(rvec, h + NBUF, h)
            for h in range(NBUF):
                extract_blk(rvec, h + NBUF, h, jcol)
            return carry

        lax.fori_loop(0, CHC // 16, grp_body, 0)
        qcp.wait()
        zcp.wait()

        def dot_body(gv, ssq):
            s = pl.multiple_of(gv * 16, 16)
            gl = pl.multiple_of(k * CHC + gv * 16, 16)
            cv = cols_v[pl.ds(gl, 16)]
            half_off = jnp.bitwise_and(cv, 1) * DD
            item = lane + s
            acc_rat = jnp.zeros((16,), jnp.float32)
            acc_rel = jnp.zeros((16,), jnp.float32)
            for f in range(DD):
                p = p_v[f, pl.ds(s, 16)]
                q = plsc.load_gather(qraw, [item, half_off + f])
                z = plsc.load_gather(zraw, [item, half_off + f])
                acc_rat = acc_rat + p * q
                acc_rel = acc_rel + p * z
                ssq = ssq + p * p + q * q + z * z
            drat_v[pl.ds(gl, 16)] = acc_rat
            drel_v[pl.ds(gl, 16)] = acc_rel
            return ssq

        return lax.fori_loop(0, CHC // 16, dot_body, ssq_in)

    ssq_total = lax.fori_loop(0, NCHK, chunk_body,
                              jnp.zeros((16,), jnp.float32))

    ssq_v[...] = ssq_total
    pltpu.sync_copy(drat_v, drat_out.at[pl.ds(base, BPW)])
    pltpu.sync_copy(drel_v, drel_out.at[pl.ds(base, BPW)])
    pltpu.sync_copy(ssq_v, ssq_out.at[wid])


@functools.cache
def _sc_kernel():
    mesh = plsc.VectorSubcoreMesh(
        core_axis_name="c", subcore_axis_name="s", num_cores=NC, num_subcores=NS
    )
    return pl.kernel(
        _sc_body,
        out_type=[
            jax.ShapeDtypeStruct((BB,), jnp.float32),
            jax.ShapeDtypeStruct((BB,), jnp.float32),
            jax.ShapeDtypeStruct((NW, 16), jnp.float32),
        ],
        mesh=mesh,
        scratch_types=[
            pltpu.VMEM((BPW,), jnp.int32),
            pltpu.VMEM((BPW,), jnp.int32),
            pltpu.VMEM((NCHK, CHC), jnp.int32),
            pltpu.VMEM((DD, 128), jnp.float32),
            pltpu.VMEM((NBUF, DD, 128), jnp.float32),
            pltpu.VMEM((CHC, 128), jnp.float32),
            pltpu.VMEM((CHC, 128), jnp.float32),
            pltpu.VMEM((DD, CHC), jnp.float32),
            pltpu.VMEM((BPW,), jnp.float32),
            pltpu.VMEM((BPW,), jnp.float32),
            pltpu.VMEM((16,), jnp.float32),
            pltpu.SemaphoreType.DMA,
        ] + [pltpu.SemaphoreType.DMA] * NBUF,
        compiler_params=pltpu.CompilerParams(use_tc_tiling_on_sc=True,
                                             needs_layout_passes=False),
    )


def _loss_body(drat_ref, drel_ref, ssq_ref, rat_ref, rel_ref, sen_ref,
               senh_ref, out_ref):
    drat = drat_ref[...]
    drel = drel_ref[...]
    relv = rel_ref[...] - 0.0001
    senh = senh_ref[...] - 0.0001
    pos = relv != 0.0
    neg = (relv - 1.0) != 0.0

    s_rat = 1.0 / (1.0 + jnp.exp(-drat))
    s_rel = 1.0 / (1.0 + jnp.exp(-drel))

    loss_rel = (jnp.sum(jnp.where(pos, jnp.log(s_rel), 0.0))
                + jnp.sum(jnp.where(neg, jnp.log(1.0 - s_rel), 0.0)))

    sen_rel_val = jnp.where(senh >= EPS, 1.0, 0.0)
    flag = sen_rel_val + relv
    rel_rel = jnp.where(flag == 2.0, 1.0, 0.0)
    sen_w = sen_rel_val + (WS - 1.0) * rel_rel
    rat_w = relv + (-WS) * rel_rel

    loss_rat = jnp.sum((s_rat - rat_ref[...]) ** 2 * rat_w)
    loss_sen = jnp.sum((s_rat - sen_ref[...]) ** 2 * sen_w)
    ssq = jnp.sum(ssq_ref[...])

    part = loss_sen + loss_rat - WR * loss_rel + REG * ssq
    out_ref[...] = jnp.full((1, 1), part, dtype=jnp.float32)


def _tc_loss(drat, drel, ssq, ratval, relval, senval, senh):
    return pl.pallas_call(
        _loss_body,
        out_shape=jax.ShapeDtypeStruct((1, 1), jnp.float32),
    )(drat, drel, ssq, ratval, relval, senval, senh)


def kernel(rows, cols, ratval, relval, senval, senhelval, P, Q, Z):
    ptail = jnp.pad(P[PTAIL:, :], ((0, 0), (0, 128 - DD)))
    drat, drel, ssq = _sc_kernel()(
        rows, cols, P.T, Q.reshape(50000, 128), Z.reshape(50000, 128), ptail)
    loss = _tc_loss(
        drat.reshape(128, 128), drel.reshape(128, 128), ssq,
        ratval.reshape(128, 128), relval.reshape(128, 128),
        senval.reshape(128, 128), senhelval.reshape(128, 128),
    )
    return loss[0, 0]


# speedup vs baseline: 2.0888x; 1.0984x over previous
import functools

import jax
import jax.numpy as jnp
import numpy as np
from jax import lax
from jax.experimental import pallas as pl
from jax.experimental.pallas import tpu as pltpu
from jax.experimental.pallas import tpu_sc as plsc

NC, NS = 2, 16
NW = NC * NS
BB = 16384
DD = 64
BPW = BB // NW
CHC = 128
NCHK = BPW // CHC
NPB = 7813
PTAIL = (NPB - 1) * 128

WS = 0.7
WR = 0.3
EPS = 0.5
REG = 0.001


NBUF = 8


def _sc_body(rows_hbm, cols_hbm, pt_hbm, q2_hbm, z2_hbm, ptail_hbm,
             drat_out, drel_out, ssq_out,
             rows_v, cols_v, qidx, tailbuf, blk, qraw, zraw, p_v,
             drat_v, drel_v, ssq_v, sem, *bsems):
    wid = lax.axis_index("s") * NC + lax.axis_index("c")
    base = wid * BPW
    lane = lax.iota(jnp.int32, 16)
    pltpu.sync_copy(rows_hbm.at[pl.ds(base, BPW)], rows_v)
    pltpu.sync_copy(cols_hbm.at[pl.ds(base, BPW)], cols_v)
    pltpu.sync_copy(ptail_hbm, tailbuf)

    def qidx_body(v, carry):
        s = pl.multiple_of(v * 16, 16)
        cv = cols_v[pl.ds(s, 16)]
        k = v // (CHC // 16)
        m = v % (CHC // 16)
        qidx[k, pl.ds(m * 16, 16)] = lax.shift_right_logical(cv, 1)
        return carry

    lax.fori_loop(0, BPW // 16, qidx_body, 0)

    def issue_blk(rvec, h, slot):
        r = rvec[h]
        t = jnp.minimum(lax.shift_right_logical(r, 7), jnp.int32(NPB - 2))
        pltpu.async_copy(pt_hbm.at[:, pl.ds(t * 128, 128)], blk.at[slot],
                         bsems[slot])

    def extract_blk(rvec, h, slot, jcol):
        pltpu.make_async_copy(pt_hbm.at[:, pl.ds(0, 128)], blk.at[slot],
                              bsems[slot]).wait()
        r = rvec[h]
        is_tail = r >= PTAIL
        lm = jnp.where(is_tail, jnp.int32(0), jnp.bitwise_and(r, 127))
        lt = jnp.where(is_tail, r - PTAIL, jnp.int32(0))
        col = jnp.broadcast_to(jcol + h, (16,)).astype(jnp.int32)
        tmask = jnp.broadcast_to(is_tail, (16,))
        for gg in range(DD // 16):
            fidx = lane + (gg * 16)
            vals = plsc.load_gather(
                blk.at[slot], [fidx, jnp.broadcast_to(lm, (16,))])
            tvals = plsc.load_gather(
                tailbuf, [fidx, jnp.broadcast_to(lt, (16,))])
            plsc.store_scatter(p_v, [fidx, col],
                               jnp.where(tmask, tvals, vals))

    rvec0 = rows_v[pl.ds(0, 16)]
    for h in range(NBUF):
        issue_blk(rvec0, h, h)

    def chunk_body(k, ssq_in):
        qcp = pltpu.async_copy(q2_hbm.at[qidx.at[k]], qraw, sem)
        zcp = pltpu.async_copy(z2_hbm.at[qidx.at[k]], zraw, sem)

        def grp_body(gv, carry):
            s16 = pl.multiple_of(k * CHC + gv * 16, 16)
            rvec = rows_v[pl.ds(s16, 16)]
            s16n = jnp.minimum(s16 + 16, jnp.int32(BPW - 16))
            rvecn = rows_v[pl.ds(s16n, 16)]
            jcol = gv * 16
            for h in range(NBUF):
                extract_blk(rvec, h, h, jcol)
                issue_blk(rvec, h + NBUF, h)
            for h in range(NBUF):
                extract_blk(rvec, h + NBUF, h, jcol)
                issue_blk(rvecn, h, h)
            return carry

        lax.fori_loop(0, CHC // 16, grp_body, 0)
        qcp.wait()
        zcp.wait()

        def dot_body(gv, ssq):
            s = pl.multiple_of(gv * 16, 16)
            gl = pl.multiple_of(k * CHC + gv * 16, 16)
            cv = cols_v[pl.ds(gl, 16)]
            half_off = jnp.bitwise_and(cv, 1) * DD
            item = lane + s
            acc_rat = jnp.zeros((16,), jnp.float32)
            acc_rel = jnp.zeros((16,), jnp.float32)
            for f in range(DD):
                p = p_v[f, pl.ds(s, 16)]
                q = plsc.load_gather(qraw, [item, half_off + f])
                z = plsc.load_gather(zraw, [item, half_off + f])
                acc_rat = acc_rat + p * q
                acc_rel = acc_rel + p * z
                ssq = ssq + p * p + q * q + z * z
            drat_v[pl.ds(gl, 16)] = acc_rat
            drel_v[pl.ds(gl, 16)] = acc_rel
            return ssq

        return lax.fori_loop(0, CHC // 16, dot_body, ssq_in)

    ssq_total = lax.fori_loop(0, NCHK, chunk_body,
                              jnp.zeros((16,), jnp.float32))

    for h in range(NBUF):
        pltpu.make_async_copy(pt_hbm.at[:, pl.ds(0, 128)], blk.at[h],
                              bsems[h]).wait()

    ssq_v[...] = ssq_total
    pltpu.sync_copy(drat_v, drat_out.at[pl.ds(base, BPW)])
    pltpu.sync_copy(drel_v, drel_out.at[pl.ds(base, BPW)])
    pltpu.sync_copy(ssq_v, ssq_out.at[wid])


@functools.cache
def _sc_kernel():
    mesh = plsc.VectorSubcoreMesh(
        core_axis_name="c", subcore_axis_name="s", num_cores=NC, num_subcores=NS
    )
    return pl.kernel(
        _sc_body,
        out_type=[
            jax.ShapeDtypeStruct((BB,), jnp.float32),
            jax.ShapeDtypeStruct((BB,), jnp.float32),
            jax.ShapeDtypeStruct((NW, 16), jnp.float32),
        ],
        mesh=mesh,
        scratch_types=[
            pltpu.VMEM((BPW,), jnp.int32),
            pltpu.VMEM((BPW,), jnp.int32),
            pltpu.VMEM((NCHK, CHC), jnp.int32),
            pltpu.VMEM((DD, 128), jnp.float32),
            pltpu.VMEM((NBUF, DD, 128), jnp.float32),
            pltpu.VMEM((CHC, 128), jnp.float32),
            pltpu.VMEM((CHC, 128), jnp.float32),
            pltpu.VMEM((DD, CHC), jnp.float32),
            pltpu.VMEM((BPW,), jnp.float32),
            pltpu.VMEM((BPW,), jnp.float32),
            pltpu.VMEM((16,), jnp.float32),
            pltpu.SemaphoreType.DMA,
        ] + [pltpu.SemaphoreType.DMA] * NBUF,
        compiler_params=pltpu.CompilerParams(use_tc_tiling_on_sc=True,
                                             needs_layout_passes=False),
    )


def _loss_body(drat_ref, drel_ref, ssq_ref, rat_ref, rel_ref, sen_ref,
               senh_ref, out_ref):
    drat = drat_ref[...]
    drel = drel_ref[...]
    relv = rel_ref[...] - 0.0001
    senh = senh_ref[...] - 0.0001
    pos = relv != 0.0
    neg = (relv - 1.0) != 0.0

    s_rat = 1.0 / (1.0 + jnp.exp(-drat))
    s_rel = 1.0 / (1.0 + jnp.exp(-drel))

    loss_rel = (jnp.sum(jnp.where(pos, jnp.log(s_rel), 0.0))
                + jnp.sum(jnp.where(neg, jnp.log(1.0 - s_rel), 0.0)))

    sen_rel_val = jnp.where(senh >= EPS, 1.0, 0.0)
    flag = sen_rel_val + relv
    rel_rel = jnp.where(flag == 2.0, 1.0, 0.0)
    sen_w = sen_rel_val + (WS - 1.0) * rel_rel
    rat_w = relv + (-WS) * rel_rel

    loss_rat = jnp.sum((s_rat - rat_ref[...]) ** 2 * rat_w)
    loss_sen = jnp.sum((s_rat - sen_ref[...]) ** 2 * sen_w)
    ssq = jnp.sum(ssq_ref[...])

    part = loss_sen + loss_rat - WR * loss_rel + REG * ssq
    out_ref[...] = jnp.full((1, 1), part, dtype=jnp.float32)


def _tc_loss(drat, drel, ssq, ratval, relval, senval, senh):
    return pl.pallas_call(
        _loss_body,
        out_shape=jax.ShapeDtypeStruct((1, 1), jnp.float32),
    )(drat, drel, ssq, ratval, relval, senval, senh)


def kernel(rows, cols, ratval, relval, senval, senhelval, P, Q, Z):
    ptail = jnp.pad(P[PTAIL:, :], ((0, 0), (0, 128 - DD)))
    drat, drel, ssq = _sc_kernel()(
        rows, cols, P.T, Q.reshape(50000, 128), Z.reshape(50000, 128), ptail)
    loss = _tc_loss(
        drat.reshape(128, 128), drel.reshape(128, 128), ssq,
        ratval.reshape(128, 128), relval.reshape(128, 128),
        senval.reshape(128, 128), senhelval.reshape(128, 128),
    )
    return loss[0, 0]
